# baseline (speedup 1.0000x reference)
import jax
import jax.numpy as jnp
from jax import lax
from jax.experimental import pallas as pl
from jax.experimental.pallas import tpu as pltpu
from jax.experimental.pallas import tpu_sc as plsc

_B = 16384
_D = 16
_V = 1000000
_NC = 2
_NS = 16
_NW = _NC * _NS
_BPW = _B // _NW
_CH = 128
_NCH = _BPW // _CH
_R = _D * _NCH
_NV = 16


def _gather_body(tu, ti, fidx, out, flat, rows, rb0, rb1, d0, d1, s0, s1):
    wid = lax.axis_index("s") * _NC + lax.axis_index("c")
    pltpu.sync_copy(fidx.at[wid], flat)

    def build(c, rb):
        fr = flat.at[c]
        for m in range(_CH // _NV):
            v = fr[pl.ds(_NV * m, _NV)]
            rb[pl.ds(_NV * m, _NV)] = lax.shift_right_logical(v, 7)

    def extract(c, dst):
        fr = flat.at[c]
        rr = rows.at[c]
        for m in range(_CH // _NV):
            f16 = fr[pl.ds(_NV * m, _NV)]
            lane = lax.bitwise_and(f16, 127)
            pos = lax.iota(jnp.int32, _NV) + _NV * m
            rr[pl.ds(_NV * m, _NV)] = plsc.load_gather(dst, [pos, lane])

    for t, tbl in ((0, tu), (1, ti)):
        toff = t * _R

        def start(rb, dst, sem):
            return pltpu.async_copy(tbl.at[rb], dst, sem)

        def wait(rb, dst, sem):
            pltpu.make_async_copy(tbl.at[rb], dst, sem).wait()

        build(toff, rb0)
        start(rb0, d0, s0)

        def step(rr, carry):
            c0 = toff + 2 * rr
            build(c0 + 1, rb1)
            start(rb1, d1, s1)
            wait(rb0, d0, s0)
            extract(c0, d0)
            build(c0 + 2, rb0)
            start(rb0, d0, s0)
            wait(rb1, d1, s1)
            extract(c0 + 1, d1)
            return carry

        lax.fori_loop(0, _R // 2 - 1, step, 0)
        c = toff + _R - 2
        build(c + 1, rb1)
        start(rb1, d1, s1)
        wait(rb0, d0, s0)
        extract(c, d0)
        wait(rb1, d1, s1)
        extract(c + 1, d1)

    pltpu.sync_copy(rows, out.at[wid])


def _sc_gather(tu, ti, fidx):
    mesh = plsc.VectorSubcoreMesh(core_axis_name="c", subcore_axis_name="s")
    f = pl.kernel(
        _gather_body,
        out_type=jax.ShapeDtypeStruct((_NW, 2 * _R, _CH), jnp.float32),
        mesh=mesh,
        scratch_types=[
            pltpu.VMEM((2 * _R, _CH), jnp.int32),
            pltpu.VMEM((2 * _R, _CH), jnp.float32),
            pltpu.VMEM((_CH,), jnp.int32),
            pltpu.VMEM((_CH,), jnp.int32),
            pltpu.VMEM((_CH, 128), jnp.float32),
            pltpu.VMEM((_CH, 128), jnp.float32),
            pltpu.SemaphoreType.DMA,
            pltpu.SemaphoreType.DMA,
        ],
        compiler_params=pltpu.CompilerParams(needs_layout_passes=False),
    )
    return f(tu, ti, fidx)


def _mlp_body(x_ref, w1_ref, b1_ref, gamma_ref, beta_ref, w2_ref, b2_ref,
              out_ref):
    x = x_ref[...]
    h = lax.dot_general(w1_ref[...], x, (((0,), (0,)), ((), ())),
                        preferred_element_type=jnp.float32)
    h = h + b1_ref[...]
    h = jnp.maximum(h, 0.0)
    mean = jnp.mean(h, axis=1, keepdims=True)
    c = h - mean
    var = jnp.mean(c * c, axis=1, keepdims=True)
    hn = c * lax.rsqrt(var + 1e-5) * gamma_ref[...] + beta_ref[...]
    logit = lax.dot_general(w2_ref[...], hn, (((0,), (0,)), ((), ())),
                            preferred_element_type=jnp.float32)
    out_ref[...] = 1.0 / (1.0 + jnp.exp(-(logit + b2_ref[0])))


def _tc_mlp(x, W1, b1, gamma, beta, W2, b2):
    return pl.pallas_call(
        _mlp_body,
        out_shape=jax.ShapeDtypeStruct((1, _B), jnp.float32),
        in_specs=[
            pl.BlockSpec(memory_space=pltpu.VMEM),
            pl.BlockSpec(memory_space=pltpu.VMEM),
            pl.BlockSpec(memory_space=pltpu.VMEM),
            pl.BlockSpec(memory_space=pltpu.VMEM),
            pl.BlockSpec(memory_space=pltpu.VMEM),
            pl.BlockSpec(memory_space=pltpu.VMEM),
            pl.BlockSpec(memory_space=pltpu.SMEM),
        ],
        out_specs=pl.BlockSpec(memory_space=pltpu.VMEM),
    )(x, W1, b1.reshape(_D, 1), gamma.reshape(_D, 1), beta.reshape(_D, 1),
      W2, b2)


def kernel(user_id, item_id, user_table, item_table, W1, b1, gamma, beta,
           W2, b2):
    tuf = user_table.T.reshape(_D * _V // 128, 128)
    tif = item_table.T.reshape(_D * _V // 128, 128)
    doff = (jnp.arange(_D, dtype=jnp.int32) * _V)[None, None, :, None, None]
    ids = jnp.stack([user_id, item_id]).reshape(2, _NW, _NCH, _CH)
    fidx = (ids.transpose(1, 0, 2, 3)[:, :, None] + doff).reshape(
        _NW, 2 * _R, _CH)
    g = _sc_gather(tuf, tif, fidx)
    x = (g.reshape(_NW, 2 * _D, _NCH, _CH)
          .transpose(1, 0, 2, 3)
          .reshape(2 * _D, _B))
    y = _tc_mlp(x, W1, b1, gamma, beta, W2, b2)
    return y.reshape(_B, 1)


# speedup vs baseline: 1.0015x; 1.0015x over previous
import jax
import jax.numpy as jnp
from jax import lax
from jax.experimental import pallas as pl
from jax.experimental.pallas import tpu as pltpu
from jax.experimental.pallas import tpu_sc as plsc

_B = 16384
_D = 16
_V = 1000000
_NC = 2
_NS = 16
_NW = _NC * _NS
_BPW = _B // _NW
_CH = 32
_NCH = _BPW // _CH
_R = _D * _NCH
_NV = 16
_NSLOT = 8


def _gather_body(tu, ti, fidx, out, flat, rows, *slots):
    rbs = slots[:_NSLOT]
    dsts = slots[_NSLOT:2 * _NSLOT]
    sems = slots[2 * _NSLOT:]
    wid = lax.axis_index("s") * _NC + lax.axis_index("c")

    def build(c, rb):
        fr = flat.at[c]
        for m in range(_CH // _NV):
            v = fr[pl.ds(_NV * m, _NV)]
            rb[pl.ds(_NV * m, _NV)] = lax.shift_right_logical(v, 7)

    def extract(c, cr, dst):
        fr = flat.at[c]
        rr = rows.at[cr]
        for m in range(_CH // _NV):
            f16 = fr[pl.ds(_NV * m, _NV)]
            lane = lax.bitwise_and(f16, 127)
            pos = lax.iota(jnp.int32, _NV) + _NV * m
            rr[pl.ds(_NV * m, _NV)] = plsc.load_gather(dst, [pos, lane])

    for t, tbl in ((0, tu), (1, ti)):
        toff = 0
        pltpu.sync_copy(fidx.at[wid, t], flat)

        for s in range(_NSLOT):
            build(toff + s, rbs[s])
            pltpu.async_copy(tbl.at[rbs[s]], dsts[s], sems[s])

        def step(rr, carry):
            c0 = toff + _NSLOT * rr
            for s in range(_NSLOT):
                pltpu.make_async_copy(tbl.at[rbs[s]], dsts[s],
                                      sems[s]).wait()
                extract(c0 + s, c0 - toff + s, dsts[s])
                build(c0 + _NSLOT + s, rbs[s])
                pltpu.async_copy(tbl.at[rbs[s]], dsts[s], sems[s])
            return carry

        lax.fori_loop(0, _R // _NSLOT - 1, step, 0)
        c0 = toff + _R - _NSLOT
        for s in range(_NSLOT):
            pltpu.make_async_copy(tbl.at[rbs[s]], dsts[s], sems[s]).wait()
            extract(c0 + s, c0 - toff + s, dsts[s])
        pltpu.sync_copy(rows, out.at[wid, t])


def _sc_gather(tu, ti, fidx):
    mesh = plsc.VectorSubcoreMesh(core_axis_name="c", subcore_axis_name="s")
    scratch = [
        pltpu.VMEM((_R, _CH), jnp.int32),
        pltpu.VMEM((_R, _CH), jnp.float32),
    ]
    scratch += [pltpu.VMEM((_CH,), jnp.int32) for _ in range(_NSLOT)]
    scratch += [pltpu.VMEM((_CH, 128), jnp.float32) for _ in range(_NSLOT)]
    scratch += [pltpu.SemaphoreType.DMA for _ in range(_NSLOT)]
    f = pl.kernel(
        _gather_body,
        out_type=jax.ShapeDtypeStruct((_NW, 2, _R, _CH), jnp.float32),
        mesh=mesh,
        scratch_types=scratch,
        compiler_params=pltpu.CompilerParams(needs_layout_passes=False),
    )
    return f(tu, ti, fidx)


def _mlp_body(x_ref, w1_ref, b1_ref, gamma_ref, beta_ref, w2_ref, b2_ref,
              out_ref):
    x = x_ref[...]
    h = lax.dot_general(w1_ref[...], x, (((0,), (0,)), ((), ())),
                        preferred_element_type=jnp.float32)
    h = h + b1_ref[...]
    h = jnp.maximum(h, 0.0)
    mean = jnp.mean(h, axis=1, keepdims=True)
    c = h - mean
    var = jnp.mean(c * c, axis=1, keepdims=True)
    hn = c * lax.rsqrt(var + 1e-5) * gamma_ref[...] + beta_ref[...]
    logit = lax.dot_general(w2_ref[...], hn, (((0,), (0,)), ((), ())),
                            preferred_element_type=jnp.float32)
    out_ref[...] = 1.0 / (1.0 + jnp.exp(-(logit + b2_ref[0])))


def _tc_mlp(x, W1, b1, gamma, beta, W2, b2):
    return pl.pallas_call(
        _mlp_body,
        out_shape=jax.ShapeDtypeStruct((1, _B), jnp.float32),
        in_specs=[
            pl.BlockSpec(memory_space=pltpu.VMEM),
            pl.BlockSpec(memory_space=pltpu.VMEM),
            pl.BlockSpec(memory_space=pltpu.VMEM),
            pl.BlockSpec(memory_space=pltpu.VMEM),
            pl.BlockSpec(memory_space=pltpu.VMEM),
            pl.BlockSpec(memory_space=pltpu.VMEM),
            pl.BlockSpec(memory_space=pltpu.SMEM),
        ],
        out_specs=pl.BlockSpec(memory_space=pltpu.VMEM),
    )(x, W1, b1.reshape(_D, 1), gamma.reshape(_D, 1), beta.reshape(_D, 1),
      W2, b2)


def kernel(user_id, item_id, user_table, item_table, W1, b1, gamma, beta,
           W2, b2):
    tuf = user_table.T.reshape(_D * _V // 128, 128)
    tif = item_table.T.reshape(_D * _V // 128, 128)
    doff = (jnp.arange(_D, dtype=jnp.int32) * _V)[None, None, :, None, None]
    ids = jnp.stack([user_id, item_id]).reshape(2, _NW, _NCH, _CH)
    fidx = (ids.transpose(1, 0, 2, 3)[:, :, None] + doff).reshape(
        _NW, 2, _R, _CH)
    g = _sc_gather(tuf, tif, fidx)
    x = (g.reshape(_NW, 2 * _D, _NCH, _CH)
          .transpose(1, 0, 2, 3)
          .reshape(2 * _D, _B))
    y = _tc_mlp(x, W1, b1, gamma, beta, W2, b2)
    return y.reshape(_B, 1)


# speedup vs baseline: 4.4335x; 4.4270x over previous
import jax
import jax.numpy as jnp
from jax import lax
from jax.experimental import pallas as pl
from jax.experimental.pallas import tpu as pltpu
from jax.experimental.pallas import tpu_sc as plsc

_B = 16384
_D = 16
_V = 1000000
_NC = 2
_NS = 16
_NW = _NC * _NS
_BPW = _B // _NW
_CH = 32
_R = _BPW // _CH
_NV = 16
_NSLOT = 8


def _gather_body(tu, ti, ids, out, flat, rows, *slots):
    rbs = slots[:_NSLOT]
    dsts = slots[_NSLOT:2 * _NSLOT]
    sems = slots[2 * _NSLOT:]
    wid = lax.axis_index("s") * _NC + lax.axis_index("c")

    def build(c, rb):
        fr = flat.at[c]
        for m in range(_CH // _NV):
            v = fr[pl.ds(_NV * m, _NV)]
            rb[pl.ds(_NV * m, _NV)] = lax.shift_right_logical(v, 3)

    def extract(c, dst):
        fr = flat.at[c]
        for m in range(_CH // _NV):
            v = fr[pl.ds(_NV * m, _NV)]
            base = lax.bitwise_and(v, 7) * 16
            pos = lax.iota(jnp.int32, _NV) + _NV * m
            off = c * _CH + _NV * m
            for d in range(_D):
                rows[pl.ds(d * _BPW + off, _NV)] = plsc.load_gather(
                    dst, [pos, base + d])

    for t, tbl in ((0, tu), (1, ti)):
        pltpu.sync_copy(ids.at[wid, t], flat)

        for s in range(_NSLOT):
            build(s, rbs[s])
            pltpu.async_copy(tbl.at[rbs[s]], dsts[s], sems[s])

        def step(rr, carry):
            c0 = _NSLOT * rr
            for s in range(_NSLOT):
                pltpu.make_async_copy(tbl.at[rbs[s]], dsts[s],
                                      sems[s]).wait()
                extract(c0 + s, dsts[s])
                build(c0 + _NSLOT + s, rbs[s])
                pltpu.async_copy(tbl.at[rbs[s]], dsts[s], sems[s])
            return carry

        lax.fori_loop(0, _R // _NSLOT - 1, step, 0)
        c0 = _R - _NSLOT
        for s in range(_NSLOT):
            pltpu.make_async_copy(tbl.at[rbs[s]], dsts[s], sems[s]).wait()
            extract(c0 + s, dsts[s])
        pltpu.sync_copy(rows, out.at[wid, t])


def _sc_gather(tu, ti, ids):
    mesh = plsc.VectorSubcoreMesh(core_axis_name="c", subcore_axis_name="s")
    scratch = [
        pltpu.VMEM((_R, _CH), jnp.int32),
        pltpu.VMEM((_D * _BPW,), jnp.float32),
    ]
    scratch += [pltpu.VMEM((_CH,), jnp.int32) for _ in range(_NSLOT)]
    scratch += [pltpu.VMEM((_CH, 128), jnp.float32) for _ in range(_NSLOT)]
    scratch += [pltpu.SemaphoreType.DMA for _ in range(_NSLOT)]
    f = pl.kernel(
        _gather_body,
        out_type=jax.ShapeDtypeStruct((_NW, 2, _D * _BPW), jnp.float32),
        mesh=mesh,
        scratch_types=scratch,
        compiler_params=pltpu.CompilerParams(needs_layout_passes=False),
    )
    return f(tu, ti, ids)


def _mlp_body(x_ref, w1_ref, b1_ref, gamma_ref, beta_ref, w2_ref, b2_ref,
              out_ref):
    x = x_ref[...]
    h = lax.dot_general(w1_ref[...], x, (((0,), (0,)), ((), ())),
                        preferred_element_type=jnp.float32)
    h = h + b1_ref[...]
    h = jnp.maximum(h, 0.0)
    mean = jnp.mean(h, axis=1, keepdims=True)
    c = h - mean
    var = jnp.mean(c * c, axis=1, keepdims=True)
    hn = c * lax.rsqrt(var + 1e-5) * gamma_ref[...] + beta_ref[...]
    logit = lax.dot_general(w2_ref[...], hn, (((0,), (0,)), ((), ())),
                            preferred_element_type=jnp.float32)
    out_ref[...] = 1.0 / (1.0 + jnp.exp(-(logit + b2_ref[0])))


def _tc_mlp(x, W1, b1, gamma, beta, W2, b2):
    return pl.pallas_call(
        _mlp_body,
        out_shape=jax.ShapeDtypeStruct((1, _B), jnp.float32),
        in_specs=[
            pl.BlockSpec(memory_space=pltpu.VMEM),
            pl.BlockSpec(memory_space=pltpu.VMEM),
            pl.BlockSpec(memory_space=pltpu.VMEM),
            pl.BlockSpec(memory_space=pltpu.VMEM),
            pl.BlockSpec(memory_space=pltpu.VMEM),
            pl.BlockSpec(memory_space=pltpu.VMEM),
            pl.BlockSpec(memory_space=pltpu.SMEM),
        ],
        out_specs=pl.BlockSpec(memory_space=pltpu.VMEM),
    )(x, W1, b1.reshape(_D, 1), gamma.reshape(_D, 1), beta.reshape(_D, 1),
      W2, b2)


def kernel(user_id, item_id, user_table, item_table, W1, b1, gamma, beta,
           W2, b2):
    pu = user_table.reshape(_V // 8, 128)
    pi = item_table.reshape(_V // 8, 128)
    ids = jnp.stack([user_id, item_id]).reshape(2, _NW, _R, _CH)
    ids = ids.transpose(1, 0, 2, 3)
    g = _sc_gather(pu, pi, ids).reshape(_NW, 2, _D, _BPW)
    x = g.transpose(1, 2, 0, 3).reshape(2 * _D, _B)
    y = _tc_mlp(x, W1, b1, gamma, beta, W2, b2)
    return y.reshape(_B, 1)
